# baseline (speedup 1.0000x reference)
import functools

import jax
import jax.numpy as jnp
from jax import lax
from jax.experimental import pallas as pl
from jax.experimental.pallas import tpu as pltpu
from jax.experimental.pallas import tpu_sc as plsc

N_ROWS = 16384
C_IN = 7
C_OUT = 6
L = 16

_NC = 2
_NS = 16
NW = _NC * _NS
ROWS_PER_W = N_ROWS // NW
GROUPS = ROWS_PER_W // L
IN_PER_W = ROWS_PER_W * C_IN
OUT_PER_W = ROWS_PER_W * C_OUT

_mesh = plsc.VectorSubcoreMesh(core_axis_name="c", subcore_axis_name="s")


@functools.partial(
    pl.kernel,
    mesh=_mesh,
    out_type=jax.ShapeDtypeStruct((N_ROWS * C_OUT,), jnp.float32),
    scratch_types=[
        pltpu.VMEM((IN_PER_W,), jnp.float32),
        pltpu.VMEM((OUT_PER_W,), jnp.float32),
    ],
    compiler_params=pltpu.CompilerParams(needs_layout_passes=False),
)
def _deepjet_sc(x_hbm, out_hbm, xv, ov):
    wid = lax.axis_index("s") * _NC + lax.axis_index("c")
    pltpu.sync_copy(x_hbm.at[pl.ds(wid * IN_PER_W, IN_PER_W)], xv)

    lanes = lax.iota(jnp.int32, L)
    lanes7 = lanes * 7
    lanes6 = lanes * 6
    ones = jnp.full((L,), 1.0, jnp.float32)

    for g in range(GROUPS):
        ibase = lanes7 + g * (L * C_IN)
        obase = lanes6 + g * (L * C_OUT)
        x0 = plsc.load_gather(xv, [ibase])
        x1 = plsc.load_gather(xv, [ibase + 1])
        b = plsc.load_gather(xv, [ibase + 2])
        c = plsc.load_gather(xv, [ibase + 3])
        cvl = plsc.load_gather(xv, [ibase + 5])
        qg = plsc.load_gather(xv, [ibase + 6])
        t = c / cvl - c
        o4 = (ones - qg) * t
        o5 = qg * t
        plsc.store_scatter(ov, [obase], x0)
        plsc.store_scatter(ov, [obase + 1], x1)
        plsc.store_scatter(ov, [obase + 2], b)
        plsc.store_scatter(ov, [obase + 3], c)
        plsc.store_scatter(ov, [obase + 4], o4)
        plsc.store_scatter(ov, [obase + 5], o5)

    pltpu.sync_copy(ov, out_hbm.at[pl.ds(wid * OUT_PER_W, OUT_PER_W)])


def kernel(x):
    flat = _deepjet_sc(x.reshape(-1))
    return flat.reshape(N_ROWS, C_OUT)


# speedup vs baseline: 9.9316x; 9.9316x over previous
import functools

import jax
import jax.numpy as jnp
from jax.experimental import pallas as pl

N_ROWS = 16384
C_IN = 7
C_OUT = 6
BLK = 2048
GRID = N_ROWS // BLK


def _deepjet_body(x_ref, o_ref):
    x = x_ref[...]
    c = x[3:4, :]
    cvl = x[5:6, :]
    qg = x[6:7, :]
    t = c / cvl - c
    o_ref[...] = jnp.concatenate(
        [x[0:4, :], (1.0 - qg) * t, qg * t], axis=0
    )


@jax.jit
def _deepjet(xt):
    return pl.pallas_call(
        _deepjet_body,
        grid=(GRID,),
        in_specs=[pl.BlockSpec((C_IN, BLK), lambda i: (0, i))],
        out_specs=pl.BlockSpec((C_OUT, BLK), lambda i: (0, i)),
        out_shape=jax.ShapeDtypeStruct((C_OUT, N_ROWS), jnp.float32),
    )(xt)


def kernel(x):
    return _deepjet(x.T).T


# speedup vs baseline: 25.1928x; 2.5366x over previous
import functools

import jax
import jax.numpy as jnp
from jax.experimental import pallas as pl

N_ROWS = 16384
C_IN = 7
C_OUT = 6
BLK = 8192
GRID = N_ROWS // BLK


def _deepjet_body(x_ref, o_ref):
    x = x_ref[...]
    c = x[3:4, :]
    cvl = x[5:6, :]
    qg = x[6:7, :]
    t = c / cvl - c
    o_ref[...] = jnp.concatenate(
        [x[0:4, :], (1.0 - qg) * t, qg * t], axis=0
    )


@jax.jit
def _deepjet(xt):
    return pl.pallas_call(
        _deepjet_body,
        grid=(GRID,),
        in_specs=[pl.BlockSpec((C_IN, BLK), lambda i: (0, i))],
        out_specs=pl.BlockSpec((C_OUT, BLK), lambda i: (0, i)),
        out_shape=jax.ShapeDtypeStruct((C_OUT, N_ROWS), jnp.float32),
    )(xt)


def kernel(x):
    return _deepjet(x.T).T
